# Initial kernel scaffold; baseline (speedup 1.0000x reference)
import functools

import jax
import jax.numpy as jnp
from jax import lax
from jax.experimental import pallas as pl
from jax.experimental.pallas import tpu as pltpu
from jax.experimental.pallas import tpu_sc as plsc

_EMBED = 32
_BATCH = 4096
_HIST = 200
_B_TOTAL = _BATCH * _HIST

_INFO = plsc.get_sparse_core_info()
_NC = _INFO.num_cores
_NS = _INFO.num_subcores
_NW = _NC * _NS
_BPW = _B_TOTAL // _NW
_CH = 128
_NCH = _BPW // _CH
_K = 8
_STEPS = _NCH // _K


def _body(idx_hbm, table_hbm, out_hbm, idx_v, rows_v, sem):
    wid = lax.axis_index("s") * _NC + lax.axis_index("c")
    pltpu.sync_copy(idx_hbm.at[wid], idx_v)
    base = wid * _BPW

    def step(g, carry):
        copies = []
        for j in range(_K):
            copies.append(pltpu.async_copy(
                table_hbm.at[idx_v.at[g * _K + j]],
                rows_v.at[pl.ds(j * _CH, _CH)],
                sem,
            ))
        for c in copies:
            c.wait()
        pltpu.sync_copy(
            rows_v,
            out_hbm.at[pl.ds(base + g * (_K * _CH), _K * _CH)],
        )
        return carry

    lax.fori_loop(0, _STEPS, step, 0)


@jax.jit
def _gather(idx, table):
    mesh = plsc.VectorSubcoreMesh(core_axis_name="c", subcore_axis_name="s")
    f = pl.kernel(
        _body,
        out_type=jax.ShapeDtypeStruct((_B_TOTAL, _EMBED), jnp.float32),
        mesh=mesh,
        scratch_types=[
            pltpu.VMEM((_NCH, _CH), jnp.int32),
            pltpu.VMEM((_K * _CH, _EMBED), jnp.float32),
            pltpu.SemaphoreType.DMA,
        ],
    )
    return f(idx, table)


def kernel(tags, table):
    idx = tags.reshape(_NW, _NCH, _CH).astype(jnp.int32)
    out = _gather(idx, table)
    return out.reshape(_BATCH, _HIST, _EMBED)


# speedup vs baseline: 1.4812x; 1.4812x over previous
import functools

import jax
import jax.numpy as jnp
from jax import lax
from jax.experimental import pallas as pl
from jax.experimental.pallas import tpu as pltpu
from jax.experimental.pallas import tpu_sc as plsc

_EMBED = 32
_BATCH = 4096
_HIST = 200
_B_TOTAL = _BATCH * _HIST

_INFO = plsc.get_sparse_core_info()
_NC = _INFO.num_cores
_NS = _INFO.num_subcores
_NW = _NC * _NS
_BPW = _B_TOTAL // _NW
_CH = 128
_NCH = _BPW // _CH
_K = 8
_STEPS = _NCH // _K


def _body(idx_hbm, table_hbm, out_hbm, idx_v, rows_v, sem):
    wid = lax.axis_index("s") * _NC + lax.axis_index("c")
    pltpu.sync_copy(idx_hbm.at[wid], idx_v)
    base = wid * _BPW

    def step(g, carry):
        copies = []
        for j in range(_K):
            copies.append(pltpu.async_copy(
                table_hbm.at[idx_v.at[g * _K + j]],
                rows_v.at[pl.ds(j * _CH, _CH)],
                sem,
            ))
        for c in copies:
            c.wait()
        pltpu.sync_copy(
            rows_v,
            out_hbm.at[pl.ds(base + g * (_K * _CH), _K * _CH)],
        )
        return carry

    lax.fori_loop(0, _STEPS, step, 0)


@jax.jit
def _gather(idx, table):
    mesh = plsc.VectorSubcoreMesh(core_axis_name="c", subcore_axis_name="s")
    f = pl.kernel(
        _body,
        out_type=jax.ShapeDtypeStruct((_B_TOTAL, _EMBED), jnp.float32),
        mesh=mesh,
        compiler_params=pltpu.CompilerParams(use_tc_tiling_on_sc=False),
        scratch_types=[
            pltpu.VMEM((_NCH, _CH), jnp.int32),
            pltpu.VMEM((_K * _CH, _EMBED), jnp.float32),
            pltpu.SemaphoreType.DMA,
        ],
    )
    return f(idx, table)


def kernel(tags, table):
    idx = tags.reshape(_NW, _NCH, _CH).astype(jnp.int32)
    out = _gather(idx, table)
    return out.reshape(_BATCH, _HIST, _EMBED)


# speedup vs baseline: 1.4997x; 1.0125x over previous
import jax
import jax.numpy as jnp
from jax import lax
from jax.experimental import pallas as pl
from jax.experimental.pallas import tpu as pltpu
from jax.experimental.pallas import tpu_sc as plsc

_EMBED = 32
_BATCH = 4096
_HIST = 200
_B_TOTAL = _BATCH * _HIST

_INFO = plsc.get_sparse_core_info()
_NC = _INFO.num_cores
_NS = _INFO.num_subcores
_NW = _NC * _NS
_BPW = _B_TOTAL // _NW
_CH = 128
_NCH = _BPW // _CH
_K = 5
_GR = _K * _CH
_NBUF = 4
_NGRP = _NCH // _K
_T = _NGRP // _NBUF


def _body(idx_hbm, table_hbm, out_hbm,
          idx_v, b0, b1, b2, b3, gsem, wsem):
    bufs = [b0, b1, b2, b3]
    wid = lax.axis_index("s") * _NC + lax.axis_index("c")
    pltpu.sync_copy(idx_hbm.at[wid], idx_v)
    base = wid * _BPW

    def fire(b, g):
        for j in range(_K):
            pltpu.async_copy(
                table_hbm.at[idx_v.at[g * _K + j]],
                bufs[b].at[pl.ds(j * _CH, _CH)],
                gsem.at[b],
            )

    def wait_gathers(b):
        pltpu.make_async_copy(
            out_hbm.at[pl.ds(0, _GR)], bufs[b], gsem.at[b]).wait()

    def start_write(b, g):
        pltpu.async_copy(
            bufs[b], out_hbm.at[pl.ds(base + g * _GR, _GR)], wsem.at[b])

    def wait_write(b):
        pltpu.make_async_copy(
            bufs[b], out_hbm.at[pl.ds(0, _GR)], wsem.at[b]).wait()

    for b in range(_NBUF - 1):
        fire(b, b)

    def step(t, carry):
        for b in range(_NBUF):
            g = t * _NBUF + b
            wait_gathers(b)
            start_write(b, g)
            if b == 0:
                @pl.when(t > 0)
                def _():
                    wait_write(_NBUF - 1)
                fire(_NBUF - 1, g + _NBUF - 1)
            else:
                wait_write(b - 1)

                @pl.when(t < _T - 1)
                def _():
                    fire(b - 1, g + _NBUF - 1)
        return carry

    lax.fori_loop(0, _T, step, 0)
    wait_write(_NBUF - 1)


@jax.jit
def _gather(idx, table):
    mesh = plsc.VectorSubcoreMesh(core_axis_name="c", subcore_axis_name="s")
    f = pl.kernel(
        _body,
        out_type=jax.ShapeDtypeStruct((_B_TOTAL, _EMBED), jnp.float32),
        mesh=mesh,
        compiler_params=pltpu.CompilerParams(use_tc_tiling_on_sc=False),
        scratch_types=[
            pltpu.VMEM((_NCH, _CH), jnp.int32),
        ] + [pltpu.VMEM((_GR, _EMBED), jnp.float32) for _ in range(_NBUF)] + [
            pltpu.SemaphoreType.DMA((_NBUF,)),
            pltpu.SemaphoreType.DMA((_NBUF,)),
        ],
    )
    return f(idx, table)


def kernel(tags, table):
    idx = tags.reshape(_NW, _NCH, _CH).astype(jnp.int32)
    out = _gather(idx, table)
    return out.reshape(_BATCH, _HIST, _EMBED)
